# baseline (speedup 1.0000x reference)
import jax
import jax.numpy as jnp
from jax.experimental import pallas as pl
from jax.experimental.pallas import tpu as pltpu

_T = 128
_BLOCK_T = 4
_STEPS = _T // _BLOCK_T
_NBUF = 16
_LAG = 8


def _copy_body(x_ref, o_ref, buf, in_sems, out_sems):
    def in_copy(step, b):
        return pltpu.make_async_copy(
            x_ref.at[:, pl.ds(step * _BLOCK_T, _BLOCK_T)],
            buf.at[b],
            in_sems.at[b],
        )

    def out_copy(step, b):
        return pltpu.make_async_copy(
            buf.at[b],
            o_ref.at[:, pl.ds(step * _BLOCK_T, _BLOCK_T)],
            out_sems.at[b],
        )

    waited_out = set()
    for b in range(min(_NBUF, _STEPS)):
        in_copy(b, b).start()
    for s in range(_STEPS):
        in_copy(s, s % _NBUF).wait()
        out_copy(s, s % _NBUF).start()
        t = s - _LAG
        if t >= 0 and t + _NBUF < _STEPS:
            out_copy(t, t % _NBUF).wait()
            waited_out.add(t)
            in_copy(t + _NBUF, t % _NBUF).start()
    for t in range(_STEPS):
        if t not in waited_out:
            out_copy(t, t % _NBUF).wait()


def kernel(x):
    n, t, d, h, w = x.shape
    out = pl.pallas_call(
        _copy_body,
        out_shape=jax.ShapeDtypeStruct(x.shape, x.dtype),
        in_specs=[pl.BlockSpec(memory_space=pl.MemorySpace.ANY)],
        out_specs=pl.BlockSpec(memory_space=pl.MemorySpace.ANY),
        scratch_shapes=[
            pltpu.VMEM((_NBUF, n, _BLOCK_T, d, h, w), jnp.float32),
            pltpu.SemaphoreType.DMA((_NBUF,)),
            pltpu.SemaphoreType.DMA((_NBUF,)),
        ],
    )(x)
    return out
